# baseline (speedup 1.0000x reference)
import dataclasses
import functools

import jax
import jax.numpy as jnp
from jax import lax
from jax.experimental import pallas as pl
from jax.experimental.pallas import tpu as pltpu
from jax.experimental.pallas import tpu_sc as plsc

B = 128
Q = 900
C = 80
N = Q * C
K = 300
KPAD = 304
L = 16
NV = N // L
NHALF = N // 2
NVH = NHALF // L
NP = 72192
NVP = NP // L
NGR = NVP // L
CAP = 4096
CAP2 = 512
NCORES = 2
NSUB = 16
NW = NCORES * NSUB
NCHUNK = 1
BCH = B // NCHUNK
ROWS_PER_W = BCH // NW
I32MIN = -2147483648
SPEC_THR = 2.0


def _mono(bits):
    return bits ^ (lax.shift_right_arithmetic(bits, 31) | jnp.int32(I32MIN))


def _sc_body(logits_hbm, boxes_hbm, lab_hbm, box_hbm, sco_hbm,
             row_a, row_b, boxrow_v, hist_v, shist_v,
             cand_f, cand_i, cand2_k, cand2_i, sort_k, sort_i,
             out_lab, out_box, out_sco, sem_a, sem_b):
    wid = lax.axis_index("c") * NSUB + lax.axis_index("s")
    lanes = lax.iota(jnp.int32, L)
    ones = jnp.ones((L,), jnp.int32)
    zeros = jnp.zeros((L,), jnp.int32)
    lane0 = lanes == 0
    dbias = plsc.scan_count(zeros)[0][0]

    r0 = wid * ROWS_PER_W
    pltpu.async_copy(logits_hbm.at[r0, 0], row_a, sem_a)
    pltpu.async_copy(logits_hbm.at[r0, 1], row_b, sem_b)

    @pl.loop(0, ROWS_PER_W)
    def _row(k):
        r = wid * ROWS_PER_W + k
        pltpu.sync_copy(boxes_hbm.at[r], boxrow_v)

        pltpu.make_async_copy(logits_hbm.at[r, 0], row_a, sem_a).wait()

        @plsc.parallel_loop(0, NVH, carry=jnp.int32(0), unroll=16)
        def _scan_a(i, off):
            x = row_a[pl.ds(i * L, L)]
            m = x >= SPEC_THR
            idxv = i * L + lanes
            plsc.store_compressed(cand_i.at[pl.ds(off, L)], idxv, mask=m)
            cnt = plsc.all_reduce_population_count(m)[0]
            return jnp.minimum(off + cnt, CAP)

        pltpu.make_async_copy(logits_hbm.at[r, 1], row_b, sem_b).wait()

        @plsc.parallel_loop(0, NVH, carry=_scan_a, unroll=16)
        def _scan_b(i, off):
            x = row_b[pl.ds(i * L, L)]
            m = x >= SPEC_THR
            idxv = NHALF + i * L + lanes
            plsc.store_compressed(cand_i.at[pl.ds(off, L)], idxv, mask=m)
            cnt = plsc.all_reduce_population_count(m)[0]
            return jnp.minimum(off + cnt, CAP)

        n_spec = _scan_b

        def _fastpath(_):
            return n_spec

        def _fallback(_):
            @pl.loop(0, 256)
            def _z(j):
                hist_v[pl.ds(j * L, L)] = zeros

            for buf in (row_a, row_b):
                @pl.loop(0, NVH)
                def _p1(i):
                    x = buf[pl.ds(i * L, L)]
                    mono = _mono(plsc.bitcast(x, jnp.int32))
                    digit = lax.shift_right_logical(mono, 24)
                    addr = lax.shift_left(digit, 4) | lanes
                    plsc.addupdate_scatter(hist_v, [addr], ones)

            def _w_cond(c):
                return c[1] < K

            def _w_body(c):
                d, acc = c
                d = d - 1
                t = jnp.sum(hist_v[pl.ds(d * L, L)])
                return (d, acc + t)

            d0, _ = lax.while_loop(
                _w_cond, _w_body, (jnp.int32(256), jnp.int32(0)))

            off = jnp.int32(0)
            for buf, base in ((row_a, 0), (row_b, NHALF)):
                def _p2(i, o, buf=buf, base=base):
                    x = buf[pl.ds(i * L, L)]
                    mono = _mono(plsc.bitcast(x, jnp.int32))
                    msk = lax.shift_right_logical(mono, 24) >= d0
                    idxv = base + i * L + lanes
                    plsc.store_compressed(cand_i.at[pl.ds(o, L)], idxv,
                                          mask=msk)
                    cnt = plsc.all_reduce_population_count(msk)[0]
                    return jnp.minimum(o + cnt, CAP)

                off = lax.fori_loop(0, NVH, _p2, off)
            return off

        ok = (n_spec >= K) & (n_spec < CAP)
        n_cand = lax.cond(ok, _fastpath, _fallback, None)
        nv2 = lax.div(n_cand + (L - 1), jnp.int32(L))

        def _walk_cond(c):
            return c[1] < K

        def _boundary(acc_init):
            def _wa(c):
                g, acc, _ = c
                g = g - 1
                t = jnp.sum(shist_v[pl.ds(g * L, L)])
                return (g, acc + t, t)

            g0, accA, tg = lax.while_loop(
                _walk_cond, _wa, (jnp.int32(16), acc_init, jnp.int32(0)))

            def _wb(c):
                d, acc, _ = c
                d = d - 1
                t = jnp.sum(hist_v[pl.ds(d * L, L)])
                return (d, acc + t, t)

            return lax.while_loop(
                _walk_cond, _wb,
                (g0 * L + L, accA - tg, jnp.int32(0)))

        @plsc.parallel_loop(0, 256, unroll=8)
        def _z0(j):
            hist_v[pl.ds(j * L, L)] = zeros

        for j in range(16):
            shist_v[pl.ds(j * L, L)] = zeros

        @plsc.parallel_loop(0, nv2, unroll=4)
        def _h0(j):
            valid = (j * L + lanes) < n_cand
            idxg = cand_i[pl.ds(j * L, L)]
            in_a = idxg < NHALF
            ia = jnp.clip(idxg, 0, NHALF - 1)
            ib = jnp.clip(idxg - NHALF, 0, NHALF - 1)
            xa = plsc.load_gather(row_a, [ia], mask=valid & in_a)
            xb = plsc.load_gather(row_b, [ib], mask=valid & (~in_a))
            x = jnp.where(in_a, xa, xb)
            mono = _mono(plsc.bitcast(x, jnp.int32))
            cand_f[pl.ds(j * L, L)] = plsc.bitcast(mono, jnp.float32)
            digit = lax.shift_right_logical(mono, 24)
            addr = lax.shift_left(digit, 4) | lanes
            plsc.addupdate_scatter(hist_v, [addr], ones, mask=valid)
            addrc = (digit & 0xF0) | lanes
            plsc.addupdate_scatter(shist_v, [addrc], ones, mask=valid)

        @pl.when(k + 1 < ROWS_PER_W)
        def _prefetch():
            rn = jnp.minimum(r + 1, B - 1)
            pltpu.async_copy(logits_hbm.at[rn, 0], row_a, sem_a)
            pltpu.async_copy(logits_hbm.at[rn, 1], row_b, sem_b)

        d0, acc0, t_last = _boundary(jnp.int32(0))
        n_above0 = acc0 - t_last

        @plsc.parallel_loop(0, 256, unroll=8)
        def _z1(j):
            hist_v[pl.ds(j * L, L)] = zeros

        for j in range(16):
            shist_v[pl.ds(j * L, L)] = zeros

        @plsc.parallel_loop(0, nv2, unroll=4)
        def _h1(j):
            mono = plsc.bitcast(cand_f[pl.ds(j * L, L)], jnp.int32)
            valid = (j * L + lanes) < n_cand
            msk = (lax.shift_right_logical(mono, 24) == d0) & valid
            d1 = lax.shift_right_logical(mono, 16) & 255
            addr = lax.shift_left(d1, 4) | lanes
            plsc.addupdate_scatter(hist_v, [addr], ones, mask=msk)
            addrc = (d1 & 0xF0) | lanes
            plsc.addupdate_scatter(shist_v, [addrc], ones, mask=msk)

        d1f, _, _ = _boundary(n_above0)
        b16 = d0 * 256 + d1f

        @plsc.parallel_loop(0, nv2, carry=jnp.int32(0), unroll=4)
        def _p3(j, off):
            mono = plsc.bitcast(cand_f[pl.ds(j * L, L)], jnp.int32)
            idxv = cand_i[pl.ds(j * L, L)]
            valid = (j * L + lanes) < n_cand
            msk = (lax.shift_right_logical(mono, 16) >= b16) & valid
            plsc.store_compressed(cand2_k.at[pl.ds(off, L)], mono, mask=msk)
            plsc.store_compressed(cand2_i.at[pl.ds(off, L)], idxv, mask=msk)
            cnt = plsc.all_reduce_population_count(msk)[0]
            return jnp.minimum(off + cnt, CAP2)

        n2 = _p3
        nv3 = lax.div(n2 + (L - 1), jnp.int32(L))

        @plsc.parallel_loop(0, nv3, unroll=2)
        def _sig(j):
            mono = cand2_k[pl.ds(j * L, L)]
            t = lax.shift_right_arithmetic(mono, 31)
            bits = mono ^ ((t ^ jnp.int32(-1)) | jnp.int32(I32MIN))
            x = plsc.bitcast(bits, jnp.float32)
            s = 1.0 / (1.0 + jnp.exp(-x))
            cand2_k[pl.ds(j * L, L)] = plsc.bitcast(s, jnp.int32)

        def _mmx(j, c):
            u = cand2_k[pl.ds(j * L, L)]
            valid = (j * L + lanes) < n2
            b3 = lax.shift_right_logical(u, 24) & 255
            mn = jnp.min(jnp.where(valid, b3, 255))
            mx = jnp.max(jnp.where(valid, b3, 0))
            return (jnp.minimum(c[0], mn), jnp.maximum(c[1], mx))

        b3mn, b3mx = lax.fori_loop(0, nv3, _mmx,
                                   (jnp.int32(255), jnp.int32(0)))

        def _radix_pass(src_k, src_i, dst_k, dst_i, sh):
            @pl.loop(0, 16)
            def _zs(j):
                shist_v[pl.ds(j * L, L)] = zeros

            @pl.loop(0, nv3)
            def _hist(j):
                u = src_k[pl.ds(j * L, L)]
                valid = (j * L + lanes) < n2
                d = 255 - (lax.shift_right_logical(u, sh) & 255)
                dup, last = plsc.scan_count(d, mask=valid)
                base = plsc.load_gather(shist_v, [d])
                plsc.store_scatter(shist_v, [d], base + dup - dbias + 1,
                                   mask=valid & last)

            carry = jnp.int32(0)
            for g in range(16):
                v = shist_v[pl.ds(g * L, L)]
                incl = plsc.cumsum(v)
                shist_v[pl.ds(g * L, L)] = incl - v + carry
                carry = carry + jnp.sum(v)

            @pl.loop(0, nv3)
            def _scat(j):
                u = src_k[pl.ds(j * L, L)]
                ix = src_i[pl.ds(j * L, L)]
                valid = (j * L + lanes) < n2
                d = 255 - (lax.shift_right_logical(u, sh) & 255)
                dup, last = plsc.scan_count(d, mask=valid)
                base = plsc.load_gather(shist_v, [d])
                pos = base + dup - dbias
                plsc.store_scatter(dst_k, [pos], u, mask=valid)
                plsc.store_scatter(dst_i, [pos], ix, mask=valid)
                plsc.store_scatter(shist_v, [d], pos + 1, mask=valid & last)

        for p in range(4):
            src_k, src_i = (cand2_k, cand2_i) if p % 2 == 0 else (sort_k, sort_i)
            dst_k, dst_i = (sort_k, sort_i) if p % 2 == 0 else (cand2_k, cand2_i)
            if p < 3:
                _radix_pass(src_k, src_i, dst_k, dst_i, 8 * p)
            else:
                def _do(_):
                    _radix_pass(src_k, src_i, dst_k, dst_i, 24)
                    return jnp.int32(0)

                def _copy(_):
                    @plsc.parallel_loop(0, nv3, unroll=4)
                    def _cp(j):
                        dst_k[pl.ds(j * L, L)] = src_k[pl.ds(j * L, L)]
                        dst_i[pl.ds(j * L, L)] = src_i[pl.ds(j * L, L)]
                    return jnp.int32(0)

                lax.cond(b3mn == b3mx, _copy, _do, None)

        @plsc.parallel_loop(0, KPAD // L, unroll=2)
        def _out(j):
            sbits = cand2_k[pl.ds(j * L, L)]
            idxv = cand2_i[pl.ds(j * L, L)]
            score = plsc.bitcast(sbits, jnp.float32)
            qidx = lax.div(idxv, jnp.int32(C))
            labv = idxv - qidx * C
            qc = jnp.clip(qidx, 0, Q - 1)
            a0 = qc * 4
            cx = plsc.load_gather(boxrow_v, [a0])
            cy = plsc.load_gather(boxrow_v, [a0 + 1])
            w = plsc.load_gather(boxrow_v, [a0 + 2])
            h = plsc.load_gather(boxrow_v, [a0 + 3])
            out_sco[pl.ds(j * L, L)] = score
            out_lab[pl.ds(j * L, L)] = labv
            e4 = (j * L + lanes) * 4
            plsc.store_scatter(out_box, [e4], cx - 0.5 * w)
            plsc.store_scatter(out_box, [e4 + 1], cy - 0.5 * h)
            plsc.store_scatter(out_box, [e4 + 2], cx + 0.5 * w)
            plsc.store_scatter(out_box, [e4 + 3], cy + 0.5 * h)

        pltpu.sync_copy(out_lab, lab_hbm.at[r])
        pltpu.sync_copy(out_box, box_hbm.at[r])
        pltpu.sync_copy(out_sco, sco_hbm.at[r])


@jax.jit
def _sc_topk(logits_flat, boxes_flat):
    mesh = plsc.VectorSubcoreMesh(core_axis_name="c", subcore_axis_name="s")
    cp = pltpu.CompilerParams()
    if "needs_layout_passes" in pltpu.CompilerParams.__dataclass_fields__:
        cp = dataclasses.replace(cp, needs_layout_passes=False)
    f = pl.kernel(
        _sc_body,
        out_type=(
            jax.ShapeDtypeStruct((BCH, KPAD), jnp.int32),
            jax.ShapeDtypeStruct((BCH, KPAD * 4), jnp.float32),
            jax.ShapeDtypeStruct((BCH, KPAD), jnp.float32),
        ),
        mesh=mesh,
        scratch_types=[
            pltpu.VMEM((NHALF,), jnp.float32),
            pltpu.VMEM((NHALF,), jnp.float32),
            pltpu.VMEM((4 * Q,), jnp.float32),
            pltpu.VMEM((256 * L,), jnp.int32),
            pltpu.VMEM((256,), jnp.int32),
            pltpu.VMEM((CAP + L,), jnp.float32),
            pltpu.VMEM((CAP + L,), jnp.int32),
            pltpu.VMEM((CAP2 + L,), jnp.int32),
            pltpu.VMEM((CAP2 + L,), jnp.int32),
            pltpu.VMEM((CAP2 + L,), jnp.int32),
            pltpu.VMEM((CAP2 + L,), jnp.int32),
            pltpu.VMEM((KPAD,), jnp.int32),
            pltpu.VMEM((KPAD * 4,), jnp.float32),
            pltpu.VMEM((KPAD,), jnp.float32),
            pltpu.SemaphoreType.DMA,
            pltpu.SemaphoreType.DMA,
        ],
        compiler_params=cp,
    )
    return f(logits_flat, boxes_flat)


def kernel(pred_logits, pred_boxes):
    outs = []
    for c in range(NCHUNK):
        lo = c * BCH
        lp = pred_logits[lo:lo + BCH].reshape(BCH, 2, NHALF)
        bp = pred_boxes[lo:lo + BCH].reshape(BCH, 4 * Q)
        outs.append(_sc_topk(lp, bp))
    lab_p = jnp.concatenate([o[0] for o in outs], axis=0)
    box_p = jnp.concatenate([o[1] for o in outs], axis=0)
    sco_p = jnp.concatenate([o[2] for o in outs], axis=0)
    labels = lab_p[:, :K]
    boxes = box_p.reshape(B, KPAD, 4)[:, :K, :]
    scores = sco_p[:, :K]
    return (labels, boxes, scores)


# speedup vs baseline: 1.1176x; 1.1176x over previous
import dataclasses
import functools

import jax
import jax.numpy as jnp
from jax import lax
from jax.experimental import pallas as pl
from jax.experimental.pallas import tpu as pltpu
from jax.experimental.pallas import tpu_sc as plsc

B = 128
Q = 900
C = 80
N = Q * C
K = 300
KPAD = 304
L = 16
NV = N // L
NHALF = N // 2
NVH = NHALF // L
NP = 72192
NVP = NP // L
NGR = NVP // L
CAP = 4096
CAP2 = 512
NCORES = 2
NSUB = 16
NW = NCORES * NSUB
NCHUNK = 1
BCH = B // NCHUNK
ROWS_PER_W = BCH // NW
I32MIN = -2147483648
SPEC_THR = 2.0


def _mono(bits):
    return bits ^ (lax.shift_right_arithmetic(bits, 31) | jnp.int32(I32MIN))


def _sc_body(logits_hbm, boxes_hbm, lab_hbm, box_hbm, sco_hbm,
             row_v, boxrow_v, hist_v, shist_v,
             cand_f, cand_i, cand2_k, cand2_i, sort_k, sort_i,
             out_lab, out_box, out_sco):
    wid = lax.axis_index("c") * NSUB + lax.axis_index("s")
    lanes = lax.iota(jnp.int32, L)
    ones = jnp.ones((L,), jnp.int32)
    zeros = jnp.zeros((L,), jnp.int32)
    lane0 = lanes == 0
    dbias = plsc.scan_count(zeros)[0][0]

    @pl.loop(0, ROWS_PER_W)
    def _row(k):
        r = wid * ROWS_PER_W + k
        pltpu.sync_copy(logits_hbm.at[r], row_v)
        pltpu.sync_copy(boxes_hbm.at[r], boxrow_v)

        @plsc.parallel_loop(0, NV, carry=jnp.int32(0), unroll=16)
        def _scan(i, off):
            x = row_v[pl.ds(i * L, L)]
            m = x >= SPEC_THR
            idxv = i * L + lanes
            plsc.store_compressed(cand_i.at[pl.ds(off, L)], idxv, mask=m)
            cnt = plsc.all_reduce_population_count(m)[0]
            return jnp.minimum(off + cnt, CAP)

        n_spec = _scan

        def _fastpath(_):
            return n_spec

        def _fallback(_):
            @pl.loop(0, 256)
            def _z(j):
                hist_v[pl.ds(j * L, L)] = zeros

            @pl.loop(0, NV)
            def _p1(i):
                x = row_v[pl.ds(i * L, L)]
                mono = _mono(plsc.bitcast(x, jnp.int32))
                digit = lax.shift_right_logical(mono, 24)
                addr = lax.shift_left(digit, 4) | lanes
                plsc.addupdate_scatter(hist_v, [addr], ones)

            def _w_cond(c):
                return c[1] < K

            def _w_body(c):
                d, acc = c
                d = d - 1
                t = jnp.sum(hist_v[pl.ds(d * L, L)])
                return (d, acc + t)

            d0, _ = lax.while_loop(
                _w_cond, _w_body, (jnp.int32(256), jnp.int32(0)))

            def _p2(i, o):
                x = row_v[pl.ds(i * L, L)]
                mono = _mono(plsc.bitcast(x, jnp.int32))
                msk = lax.shift_right_logical(mono, 24) >= d0
                idxv = i * L + lanes
                plsc.store_compressed(cand_i.at[pl.ds(o, L)], idxv,
                                      mask=msk)
                cnt = plsc.all_reduce_population_count(msk)[0]
                return jnp.minimum(o + cnt, CAP)

            return lax.fori_loop(0, NV, _p2, jnp.int32(0))

        ok = (n_spec >= K) & (n_spec < CAP)
        n_cand = lax.cond(ok, _fastpath, _fallback, None)
        nv2 = lax.div(n_cand + (L - 1), jnp.int32(L))

        def _walk_cond(c):
            return c[1] < K

        def _boundary(acc_init):
            def _wa(c):
                g, acc, _ = c
                g = g - 1
                t = jnp.sum(shist_v[pl.ds(g * L, L)])
                return (g, acc + t, t)

            g0, accA, tg = lax.while_loop(
                _walk_cond, _wa, (jnp.int32(16), acc_init, jnp.int32(0)))

            def _wb(c):
                d, acc, _ = c
                d = d - 1
                t = jnp.sum(hist_v[pl.ds(d * L, L)])
                return (d, acc + t, t)

            return lax.while_loop(
                _walk_cond, _wb,
                (g0 * L + L, accA - tg, jnp.int32(0)))

        @plsc.parallel_loop(0, 256, unroll=8)
        def _z0(j):
            hist_v[pl.ds(j * L, L)] = zeros

        for j in range(16):
            shist_v[pl.ds(j * L, L)] = zeros

        @plsc.parallel_loop(0, nv2, unroll=4)
        def _h0(j):
            valid = (j * L + lanes) < n_cand
            idxg = jnp.clip(cand_i[pl.ds(j * L, L)], 0, N - 1)
            x = plsc.load_gather(row_v, [idxg], mask=valid)
            mono = _mono(plsc.bitcast(x, jnp.int32))
            cand_f[pl.ds(j * L, L)] = plsc.bitcast(mono, jnp.float32)
            digit = lax.shift_right_logical(mono, 24)
            addr = lax.shift_left(digit, 4) | lanes
            plsc.addupdate_scatter(hist_v, [addr], ones, mask=valid)
            addrc = (digit & 0xF0) | lanes
            plsc.addupdate_scatter(shist_v, [addrc], ones, mask=valid)

        d0, acc0, t_last = _boundary(jnp.int32(0))
        n_above0 = acc0 - t_last

        @plsc.parallel_loop(0, 256, unroll=8)
        def _z1(j):
            hist_v[pl.ds(j * L, L)] = zeros

        for j in range(16):
            shist_v[pl.ds(j * L, L)] = zeros

        @plsc.parallel_loop(0, nv2, unroll=4)
        def _h1(j):
            mono = plsc.bitcast(cand_f[pl.ds(j * L, L)], jnp.int32)
            valid = (j * L + lanes) < n_cand
            msk = (lax.shift_right_logical(mono, 24) == d0) & valid
            d1 = lax.shift_right_logical(mono, 16) & 255
            addr = lax.shift_left(d1, 4) | lanes
            plsc.addupdate_scatter(hist_v, [addr], ones, mask=msk)
            addrc = (d1 & 0xF0) | lanes
            plsc.addupdate_scatter(shist_v, [addrc], ones, mask=msk)

        d1f, _, _ = _boundary(n_above0)
        b16 = d0 * 256 + d1f

        @plsc.parallel_loop(0, nv2, carry=jnp.int32(0), unroll=4)
        def _p3(j, off):
            mono = plsc.bitcast(cand_f[pl.ds(j * L, L)], jnp.int32)
            idxv = cand_i[pl.ds(j * L, L)]
            valid = (j * L + lanes) < n_cand
            msk = (lax.shift_right_logical(mono, 16) >= b16) & valid
            plsc.store_compressed(cand2_k.at[pl.ds(off, L)], mono, mask=msk)
            plsc.store_compressed(cand2_i.at[pl.ds(off, L)], idxv, mask=msk)
            cnt = plsc.all_reduce_population_count(msk)[0]
            return jnp.minimum(off + cnt, CAP2)

        n2 = _p3
        nv3 = lax.div(n2 + (L - 1), jnp.int32(L))

        @plsc.parallel_loop(0, nv3, unroll=2)
        def _sig(j):
            mono = cand2_k[pl.ds(j * L, L)]
            t = lax.shift_right_arithmetic(mono, 31)
            bits = mono ^ ((t ^ jnp.int32(-1)) | jnp.int32(I32MIN))
            x = plsc.bitcast(bits, jnp.float32)
            s = 1.0 / (1.0 + jnp.exp(-x))
            cand2_k[pl.ds(j * L, L)] = plsc.bitcast(s, jnp.int32)

        def _mmx(j, c):
            u = cand2_k[pl.ds(j * L, L)]
            valid = (j * L + lanes) < n2
            b3 = lax.shift_right_logical(u, 24) & 255
            mn = jnp.min(jnp.where(valid, b3, 255))
            mx = jnp.max(jnp.where(valid, b3, 0))
            return (jnp.minimum(c[0], mn), jnp.maximum(c[1], mx))

        b3mn, b3mx = lax.fori_loop(0, nv3, _mmx,
                                   (jnp.int32(255), jnp.int32(0)))

        def _radix_pass(src_k, src_i, dst_k, dst_i, sh):
            @pl.loop(0, 16)
            def _zs(j):
                shist_v[pl.ds(j * L, L)] = zeros

            @pl.loop(0, nv3)
            def _hist(j):
                u = src_k[pl.ds(j * L, L)]
                valid = (j * L + lanes) < n2
                d = 255 - (lax.shift_right_logical(u, sh) & 255)
                dup, last = plsc.scan_count(d, mask=valid)
                base = plsc.load_gather(shist_v, [d])
                plsc.store_scatter(shist_v, [d], base + dup - dbias + 1,
                                   mask=valid & last)

            carry = jnp.int32(0)
            for g in range(16):
                v = shist_v[pl.ds(g * L, L)]
                incl = plsc.cumsum(v)
                shist_v[pl.ds(g * L, L)] = incl - v + carry
                carry = carry + jnp.sum(v)

            @pl.loop(0, nv3)
            def _scat(j):
                u = src_k[pl.ds(j * L, L)]
                ix = src_i[pl.ds(j * L, L)]
                valid = (j * L + lanes) < n2
                d = 255 - (lax.shift_right_logical(u, sh) & 255)
                dup, last = plsc.scan_count(d, mask=valid)
                base = plsc.load_gather(shist_v, [d])
                pos = base + dup - dbias
                plsc.store_scatter(dst_k, [pos], u, mask=valid)
                plsc.store_scatter(dst_i, [pos], ix, mask=valid)
                plsc.store_scatter(shist_v, [d], pos + 1, mask=valid & last)

        for p in range(4):
            src_k, src_i = (cand2_k, cand2_i) if p % 2 == 0 else (sort_k, sort_i)
            dst_k, dst_i = (sort_k, sort_i) if p % 2 == 0 else (cand2_k, cand2_i)
            if p < 3:
                _radix_pass(src_k, src_i, dst_k, dst_i, 8 * p)
            else:
                def _do(_):
                    _radix_pass(src_k, src_i, dst_k, dst_i, 24)
                    return jnp.int32(0)

                def _copy(_):
                    @plsc.parallel_loop(0, nv3, unroll=4)
                    def _cp(j):
                        dst_k[pl.ds(j * L, L)] = src_k[pl.ds(j * L, L)]
                        dst_i[pl.ds(j * L, L)] = src_i[pl.ds(j * L, L)]
                    return jnp.int32(0)

                lax.cond(b3mn == b3mx, _copy, _do, None)

        @plsc.parallel_loop(0, KPAD // L, unroll=2)
        def _out(j):
            sbits = cand2_k[pl.ds(j * L, L)]
            idxv = cand2_i[pl.ds(j * L, L)]
            score = plsc.bitcast(sbits, jnp.float32)
            qidx = lax.div(idxv, jnp.int32(C))
            labv = idxv - qidx * C
            qc = jnp.clip(qidx, 0, Q - 1)
            a0 = qc * 4
            cx = plsc.load_gather(boxrow_v, [a0])
            cy = plsc.load_gather(boxrow_v, [a0 + 1])
            w = plsc.load_gather(boxrow_v, [a0 + 2])
            h = plsc.load_gather(boxrow_v, [a0 + 3])
            out_sco[pl.ds(j * L, L)] = score
            out_lab[pl.ds(j * L, L)] = labv
            e4 = (j * L + lanes) * 4
            plsc.store_scatter(out_box, [e4], cx - 0.5 * w)
            plsc.store_scatter(out_box, [e4 + 1], cy - 0.5 * h)
            plsc.store_scatter(out_box, [e4 + 2], cx + 0.5 * w)
            plsc.store_scatter(out_box, [e4 + 3], cy + 0.5 * h)

        pltpu.sync_copy(out_lab, lab_hbm.at[r])
        pltpu.sync_copy(out_box, box_hbm.at[r])
        pltpu.sync_copy(out_sco, sco_hbm.at[r])


@jax.jit
def _sc_topk(logits_flat, boxes_flat):
    mesh = plsc.VectorSubcoreMesh(core_axis_name="c", subcore_axis_name="s")
    cp = pltpu.CompilerParams()
    if "needs_layout_passes" in pltpu.CompilerParams.__dataclass_fields__:
        cp = dataclasses.replace(cp, needs_layout_passes=False)
    f = pl.kernel(
        _sc_body,
        out_type=(
            jax.ShapeDtypeStruct((BCH, KPAD), jnp.int32),
            jax.ShapeDtypeStruct((BCH, KPAD * 4), jnp.float32),
            jax.ShapeDtypeStruct((BCH, KPAD), jnp.float32),
        ),
        mesh=mesh,
        scratch_types=[
            pltpu.VMEM((N,), jnp.float32),
            pltpu.VMEM((4 * Q,), jnp.float32),
            pltpu.VMEM((256 * L,), jnp.int32),
            pltpu.VMEM((256,), jnp.int32),
            pltpu.VMEM((CAP + L,), jnp.float32),
            pltpu.VMEM((CAP + L,), jnp.int32),
            pltpu.VMEM((CAP2 + L,), jnp.int32),
            pltpu.VMEM((CAP2 + L,), jnp.int32),
            pltpu.VMEM((CAP2 + L,), jnp.int32),
            pltpu.VMEM((CAP2 + L,), jnp.int32),
            pltpu.VMEM((KPAD,), jnp.int32),
            pltpu.VMEM((KPAD * 4,), jnp.float32),
            pltpu.VMEM((KPAD,), jnp.float32),
        ],
        compiler_params=cp,
    )
    return f(logits_flat, boxes_flat)


def kernel(pred_logits, pred_boxes):
    outs = []
    for c in range(NCHUNK):
        lo = c * BCH
        lp = pred_logits[lo:lo + BCH].reshape(BCH, N)
        bp = pred_boxes[lo:lo + BCH].reshape(BCH, 4 * Q)
        outs.append(_sc_topk(lp, bp))
    lab_p = jnp.concatenate([o[0] for o in outs], axis=0)
    box_p = jnp.concatenate([o[1] for o in outs], axis=0)
    sco_p = jnp.concatenate([o[2] for o in outs], axis=0)
    labels = lab_p[:, :K]
    boxes = box_p.reshape(B, KPAD, 4)[:, :K, :]
    scores = sco_p[:, :K]
    return (labels, boxes, scores)
